# baseline (speedup 1.0000x reference)
import jax
import jax.numpy as jnp
from jax import lax
from jax.experimental import pallas as pl
from jax.experimental.pallas import tpu as pltpu
from jax.experimental.pallas import tpu_sc as plsc

N = 10000
E = 160000
NS = 16
SLICE = 640
NPAD = NS * SLICE
EPT = E // NS
OCHUNK = 1024
ONT = NPAD // OCHUNK
FULL_T = N // SLICE
REM = N - FULL_T * SLICE
OFF = SLICE - REM
H1 = 2048
H2 = 1024
H3 = 256
OUT = 124
EPS = 1e-5
NEG_SLOPE = 0.01


def _sc_body(ei_h, x_h, s2_h, t_h, hacc0, hacc2, hb2, hbz, hst2,
             src_v, dst_v, tab, tab_z, x_sv,
             acc, colbuf, tdinv, twx, tz, tmp_v, tmp2, stage_v, sem, sem2):
    tid = lax.axis_index("s")
    base_e = tid * EPT
    base_n = tid * SLICE
    zeros16 = jnp.zeros((16,), jnp.float32)
    ones16 = jnp.ones((16,), jnp.float32)

    cp_src = pltpu.async_copy(ei_h.at[pl.ds(base_e, EPT)], src_v, sem)
    cp_dst = pltpu.async_copy(ei_h.at[pl.ds(E + base_e, EPT)], dst_v, sem)

    def zero(ref, lo, n):
        @plsc.parallel_loop(lo // 16, (lo + n) // 16, unroll=8)
        def _(i):
            ref[pl.ds(i * 16, 16)] = zeros16

    def reduce_cols(cb, out_ref, lo=0, hi=SLICE):
        @plsc.parallel_loop(lo // 16, hi // 16, unroll=4)
        def _(k):
            a = cb[0, pl.ds(k * 16, 16)]
            for j in range(1, NS):
                a = a + cb[j, pl.ds(k * 16, 16)]
            out_ref[pl.ds(k * 16, 16)] = a

    HA = 384
    HB = SLICE - HA

    def split_pull_reduce(hrow, out_ref):
        cpa = pltpu.async_copy(hrow.at[:, pl.ds(base_n, HA)],
                               colbuf.at[:, pl.ds(0, HA)], sem)
        cpb = pltpu.async_copy(hrow.at[:, pl.ds(base_n + HA, HB)],
                               colbuf.at[:, pl.ds(HA, HB)], sem2)
        cpa.wait()
        reduce_cols(colbuf, out_ref, 0, HA)
        cpb.wait()
        reduce_cols(colbuf, out_ref, HA, SLICE)

    def to2d(src_ref, dst2):
        for r in range(OCHUNK // 128):
            @pl.loop(0, 8)
            def _(c):
                dst2[r, pl.ds(c * 16, 16)] = src_ref[pl.ds(r * 128 + c * 16,
                                                           16)]

    zero(acc, 0, NPAD)

    @pl.when(tid < FULL_T)
    def _():
        pltpu.sync_copy(x_h.at[pl.ds(base_n, SLICE)], x_sv)

    @pl.when(tid == FULL_T)
    def _():
        pltpu.sync_copy(x_h.at[pl.ds(N - SLICE, SLICE)], tmp_v)
        for k in range(REM // 16):
            x_sv[pl.ds(k * 16, 16)] = tmp_v[pl.ds(OFF + k * 16, 16)]
        for k in range(REM // 16, SLICE // 16):
            x_sv[pl.ds(k * 16, 16)] = zeros16

    cp_src.wait()
    cp_dst.wait()

    @plsc.parallel_loop(0, EPT // 16, unroll=4)
    def _(i):
        d16 = dst_v[pl.ds(i * 16, 16)]
        plsc.addupdate_scatter(acc, [d16], ones16)

    pltpu.sync_copy(acc.at[pl.ds(0, NPAD)], hacc0.at[tid])
    plsc.subcore_barrier()
    split_pull_reduce(hacc0, tmp_v)

    @plsc.parallel_loop(0, SLICE // 16, unroll=2)
    def _(k):
        deg = tmp_v[pl.ds(k * 16, 16)] + 1.0
        i32 = plsc.bitcast(deg, jnp.int32)
        i32 = jnp.int32(0x5F3759DF) - lax.shift_right_logical(i32, 1)
        y = plsc.bitcast(i32, jnp.float32)
        half = deg * 0.5
        for _ in range(3):
            y = y * (1.5 - half * y * y)
        tdinv[pl.ds(k * 16, 16)] = y
        twx[pl.ds(k * 16, 16)] = y * x_sv[pl.ds(k * 16, 16)]

    @pl.when(tid < FULL_T)
    def _():
        pltpu.sync_copy(tdinv, hb2.at[pl.ds(base_n, SLICE)])
        pltpu.sync_copy(twx, hb2.at[pl.ds(N + base_n, SLICE)])

    @pl.when(tid == FULL_T)
    def _():
        pltpu.sync_copy(tdinv.at[pl.ds(0, REM)], hb2.at[pl.ds(base_n, REM)])
        pltpu.sync_copy(twx.at[pl.ds(0, REM)], hb2.at[pl.ds(N + base_n, REM)])

    plsc.subcore_barrier()
    cp_tab = pltpu.async_copy(hb2, tab, sem)
    zero(acc, 0, 2 * NPAD)
    cp_tab.wait()

    n16 = jnp.full((16,), N, jnp.int32)
    npad16 = jnp.full((16,), NPAD, jnp.int32)

    @plsc.parallel_loop(0, EPT // 16, unroll=4)
    def _(i):
        s16 = src_v[pl.ds(i * 16, 16)]
        d16 = dst_v[pl.ds(i * 16, 16)]
        g_d = plsc.load_gather(tab, [s16])
        g_wx = plsc.load_gather(tab, [s16 + n16])
        plsc.addupdate_scatter(acc, [d16], g_wx)
        plsc.addupdate_scatter(acc, [d16 + npad16], g_d)

    pltpu.sync_copy(acc, hacc2.at[tid])
    plsc.subcore_barrier()
    split_pull_reduce(hacc2, tmp_v)

    @plsc.parallel_loop(0, SLICE // 16, unroll=2)
    def _(k):
        dv = tdinv[pl.ds(k * 16, 16)]
        s1 = dv * (tmp_v[pl.ds(k * 16, 16)] + twx[pl.ds(k * 16, 16)])
        tz[pl.ds(k * 16, 16)] = dv * s1

    @pl.when(tid < FULL_T)
    def _():
        pltpu.sync_copy(tz, hbz.at[pl.ds(base_n, SLICE)])

    @pl.when(tid == FULL_T)
    def _():
        pltpu.sync_copy(tz.at[pl.ds(0, REM)], hbz.at[pl.ds(base_n, REM)])

    plsc.subcore_barrier()
    cp_tz = pltpu.async_copy(hbz, tab_z, sem)
    cp_t = pltpu.async_copy(hacc2.at[:, pl.ds(NPAD + base_n, SLICE)],
                            colbuf, sem)
    zero(acc, 0, NPAD)
    cp_t.wait()
    reduce_cols(colbuf, tmp_v)

    @plsc.parallel_loop(0, SLICE // 16, unroll=2)
    def _(k):
        dv = tdinv[pl.ds(k * 16, 16)]
        tmp_v[pl.ds(k * 16, 16)] = dv * (tmp_v[pl.ds(k * 16, 16)] + dv)

    pltpu.sync_copy(tmp_v, hst2.at[pl.ds(base_n, SLICE)])

    cp_tz.wait()

    @plsc.parallel_loop(0, EPT // 16, unroll=4)
    def _(i):
        s16 = src_v[pl.ds(i * 16, 16)]
        d16 = dst_v[pl.ds(i * 16, 16)]
        g_z = plsc.load_gather(tab_z, [s16])
        plsc.addupdate_scatter(acc, [d16], g_z)

    pltpu.sync_copy(acc.at[pl.ds(0, NPAD)], hacc0.at[tid])
    plsc.subcore_barrier()
    split_pull_reduce(hacc0, tmp_v)

    @plsc.parallel_loop(0, SLICE // 16, unroll=2)
    def _(k):
        dv = tdinv[pl.ds(k * 16, 16)]
        tmp_v[pl.ds(k * 16, 16)] = dv * (tmp_v[pl.ds(k * 16, 16)]
                                         + tz[pl.ds(k * 16, 16)])

    pltpu.sync_copy(tmp_v, hst2.at[pl.ds(NPAD + base_n, SLICE)])
    plsc.subcore_barrier()

    @pl.when(tid < ONT)
    def _():
        ob = tid * OCHUNK
        pltpu.sync_copy(hst2.at[pl.ds(ob, OCHUNK)], stage_v)
        to2d(stage_v, tmp2)
        pltpu.sync_copy(tmp2, t_h.at[pl.ds(tid * (OCHUNK // 128), 8), :])
        pltpu.sync_copy(hst2.at[pl.ds(NPAD + ob, OCHUNK)], stage_v)
        to2d(stage_v, tmp2)
        pltpu.sync_copy(tmp2, s2_h.at[pl.ds(tid * (OCHUNK // 128), 8), :])


_sc_graph = pl.kernel(
    _sc_body,
    out_type=(
        jax.ShapeDtypeStruct((NPAD // 128, 128), jnp.float32),
        jax.ShapeDtypeStruct((NPAD // 128, 128), jnp.float32),
        jax.ShapeDtypeStruct((NS, NPAD), jnp.float32),
        jax.ShapeDtypeStruct((NS, 2 * NPAD), jnp.float32),
        jax.ShapeDtypeStruct((2 * N,), jnp.float32),
        jax.ShapeDtypeStruct((N,), jnp.float32),
        jax.ShapeDtypeStruct((2 * NPAD,), jnp.float32),
    ),
    mesh=plsc.VectorSubcoreMesh(
        core_axis_name="c", subcore_axis_name="s", num_cores=1,
        num_subcores=NS),
    compiler_params=pltpu.CompilerParams(needs_layout_passes=False),
    scratch_types=[
        pltpu.VMEM((EPT,), jnp.int32),
        pltpu.VMEM((EPT,), jnp.int32),
        pltpu.VMEM((2 * N,), jnp.float32),
        pltpu.VMEM((N,), jnp.float32),
        pltpu.VMEM((SLICE,), jnp.float32),
        pltpu.VMEM((2 * NPAD,), jnp.float32),
        pltpu.VMEM((NS, SLICE), jnp.float32),
        pltpu.VMEM((SLICE,), jnp.float32),
        pltpu.VMEM((SLICE,), jnp.float32),
        pltpu.VMEM((SLICE,), jnp.float32),
        pltpu.VMEM((SLICE,), jnp.float32),
        pltpu.VMEM((OCHUNK // 128, 128), jnp.float32),
        pltpu.VMEM((OCHUNK,), jnp.float32),
        pltpu.SemaphoreType.DMA,
        pltpu.SemaphoreType.DMA,
    ],
)


def _uc_body(w1_ref, b1_ref, W2_ref, u_ref, c_ref):
    u_ref[...] = jnp.dot(w1_ref[...], W2_ref[...],
                         preferred_element_type=jnp.float32)
    c_ref[...] = jnp.dot(b1_ref[...], W2_ref[...],
                         preferred_element_type=jnp.float32)


_uc = pl.pallas_call(
    _uc_body,
    out_shape=(
        jax.ShapeDtypeStruct((1, H2), jnp.float32),
        jax.ShapeDtypeStruct((1, H2), jnp.float32),
    ),
)


ROWS_BLK = 2048
RB8 = ROWS_BLK // 128


def _main_body(s2f_ref, tf_ref, u_ref, c_ref, gamma_ref, beta_ref, l1w_ref,
               l1b_ref, l2w_ref, l2b_ref, s2_ref, t_ref, o_ref,
               p_s, q_s, r_s):
    i = pl.program_id(0)

    @pl.when(i == 0)
    def _():
        rows = lax.broadcasted_iota(jnp.int32, (NPAD // 128, 128), 0)
        cols = lax.broadcasted_iota(jnp.int32, (NPAD // 128, 128), 1)
        mask = (rows * 128 + cols) < N

        s2 = jnp.where(mask, s2f_ref[...], 0.0)
        t = jnp.where(mask, tf_ref[...], 0.0)
        inv_n = 1.0 / N
        m_s = jnp.sum(s2) * inv_n
        m_t = jnp.sum(t) * inv_n
        ds = jnp.where(mask, s2 - m_s, 0.0)
        dt = jnp.where(mask, t - m_t, 0.0)
        vs = jnp.sum(ds * ds) * inv_n
        vt = jnp.sum(dt * dt) * inv_n
        cv = jnp.sum(ds * dt) * inv_n

        u = u_ref[...]
        c = c_ref[...]
        var = vs * u * u + vt * c * c + 2.0 * cv * u * c
        scale = gamma_ref[...] / jnp.sqrt(var + EPS)

        p = jnp.dot(u * scale, l1w_ref[...],
                    preferred_element_type=jnp.float32)
        q = jnp.dot(c * scale, l1w_ref[...],
                    preferred_element_type=jnp.float32)
        r = jnp.dot(beta_ref[...], l1w_ref[...],
                    preferred_element_type=jnp.float32) + l1b_ref[...]
        p_s[...] = p
        q_s[...] = q
        r_s[...] = r - m_s * p - m_t * q

    na = lax.broadcasted_iota(jnp.int32, (ROWS_BLK, RB8), 0)
    ka = lax.broadcasted_iota(jnp.int32, (ROWS_BLK, RB8), 1)
    A = jnp.where(lax.shift_right_logical(na, 7) == ka, 1.0, 0.0)
    nd = lax.broadcasted_iota(jnp.int32, (ROWS_BLK, 128), 0)
    cd = lax.broadcasted_iota(jnp.int32, (ROWS_BLK, 128), 1)
    Dm = jnp.where((nd & 127) == cd, 1.0, 0.0)
    gs = jnp.dot(A, s2_ref[...], preferred_element_type=jnp.float32)
    gt = jnp.dot(A, t_ref[...], preferred_element_type=jnp.float32)
    s2c = jnp.sum(gs * Dm, axis=1, keepdims=True)
    tc = jnp.sum(gt * Dm, axis=1, keepdims=True)
    h = s2c * p_s[...] + tc * q_s[...] + r_s[...]
    h = jnp.where(h > 0, h, NEG_SLOPE * h)
    logits = jnp.dot(h, l2w_ref[...],
                     preferred_element_type=jnp.float32) + l2b_ref[...]
    m = jnp.max(logits, axis=1, keepdims=True)
    z = logits - m
    lse = jnp.log(jnp.sum(jnp.exp(z), axis=1, keepdims=True))
    o_ref[...] = z - lse


_main = pl.pallas_call(
    _main_body,
    grid=((N + ROWS_BLK - 1) // ROWS_BLK,),
    in_specs=[
        pl.BlockSpec((NPAD // 128, 128), lambda i: (0, 0)),
        pl.BlockSpec((NPAD // 128, 128), lambda i: (0, 0)),
        pl.BlockSpec((1, H2), lambda i: (0, 0)),
        pl.BlockSpec((1, H2), lambda i: (0, 0)),
        pl.BlockSpec((1, H2), lambda i: (0, 0)),
        pl.BlockSpec((1, H2), lambda i: (0, 0)),
        pl.BlockSpec((H2, H3), lambda i: (0, 0)),
        pl.BlockSpec((1, H3), lambda i: (0, 0)),
        pl.BlockSpec((H3, OUT), lambda i: (0, 0)),
        pl.BlockSpec((1, OUT), lambda i: (0, 0)),
        pl.BlockSpec((RB8, 128), lambda i: (i, 0)),
        pl.BlockSpec((RB8, 128), lambda i: (i, 0)),
    ],
    out_specs=pl.BlockSpec((ROWS_BLK, OUT), lambda i: (i, 0)),
    out_shape=jax.ShapeDtypeStruct((N, OUT), jnp.float32),
    scratch_shapes=[
        pltpu.VMEM((1, H3), jnp.float32),
        pltpu.VMEM((1, H3), jnp.float32),
        pltpu.VMEM((1, H3), jnp.float32),
    ],
)


def kernel(x, edge_index, W1, b1, W2, b2, gamma, beta, lin1_W, lin1_b,
           lin2_W, lin2_b):
    del b2
    xf = x.reshape(N).astype(jnp.float32)

    u, c = _uc(W1.reshape(1, H1), b1.reshape(1, H1), W2)
    s2p, tp, _, _, _, _, _ = _sc_graph(edge_index.reshape(2 * E), xf)

    return _main(
        s2p,
        tp,
        u,
        c,
        gamma.reshape(1, H2),
        beta.reshape(1, H2),
        lin1_W,
        lin1_b.reshape(1, H3),
        lin2_W,
        lin2_b.reshape(1, OUT),
        s2p,
        tp,
    )


# speedup vs baseline: 1.0367x; 1.0367x over previous
import jax
import jax.numpy as jnp
from jax import lax
from jax.experimental import pallas as pl
from jax.experimental.pallas import tpu as pltpu
from jax.experimental.pallas import tpu_sc as plsc

N = 10000
E = 160000
NS = 16
SLICE = 640
NPAD = NS * SLICE
EPT = E // NS
OCHUNK = 1024
ONT = NPAD // OCHUNK
FULL_T = N // SLICE
REM = N - FULL_T * SLICE
OFF = SLICE - REM
H1 = 2048
H2 = 1024
H3 = 256
OUT = 124
EPS = 1e-5
NEG_SLOPE = 0.01


def _sc_body(ei_h, x_h, s2_h, t_h, hacc0, hacc2, hb2, hbz, hst2,
             src_v, dst_v, tab, tab_z, x_sv,
             acc, colbuf, tdinv, twx, tz, tmp_v, tmp2, stage_v, sem):
    tid = lax.axis_index("s")
    base_e = tid * EPT
    base_n = tid * SLICE
    zeros16 = jnp.zeros((16,), jnp.float32)
    ones16 = jnp.ones((16,), jnp.float32)

    cp_src = pltpu.async_copy(ei_h.at[pl.ds(base_e, EPT)], src_v, sem)
    cp_dst = pltpu.async_copy(ei_h.at[pl.ds(E + base_e, EPT)], dst_v, sem)

    def zero(ref, lo, n):
        @plsc.parallel_loop(lo // 16, (lo + n) // 16, unroll=8)
        def _(i):
            ref[pl.ds(i * 16, 16)] = zeros16

    def reduce_cols(cb, out_ref):
        @plsc.parallel_loop(0, SLICE // 16, unroll=4)
        def _(k):
            a = cb[0, pl.ds(k * 16, 16)]
            for j in range(1, NS):
                a = a + cb[j, pl.ds(k * 16, 16)]
            out_ref[pl.ds(k * 16, 16)] = a

    def to2d(src_ref, dst2):
        for r in range(OCHUNK // 128):
            @pl.loop(0, 8)
            def _(c):
                dst2[r, pl.ds(c * 16, 16)] = src_ref[pl.ds(r * 128 + c * 16,
                                                           16)]

    zero(acc, 0, NPAD)

    @pl.when(tid < FULL_T)
    def _():
        pltpu.sync_copy(x_h.at[pl.ds(base_n, SLICE)], x_sv)

    @pl.when(tid == FULL_T)
    def _():
        pltpu.sync_copy(x_h.at[pl.ds(N - SLICE, SLICE)], tmp_v)
        for k in range(REM // 16):
            x_sv[pl.ds(k * 16, 16)] = tmp_v[pl.ds(OFF + k * 16, 16)]
        for k in range(REM // 16, SLICE // 16):
            x_sv[pl.ds(k * 16, 16)] = zeros16

    cp_src.wait()
    cp_dst.wait()

    @plsc.parallel_loop(0, EPT // 16, unroll=4)
    def _(i):
        d16 = dst_v[pl.ds(i * 16, 16)]
        plsc.addupdate_scatter(acc, [d16], ones16)

    pltpu.sync_copy(acc.at[pl.ds(0, NPAD)], hacc0.at[tid])
    plsc.subcore_barrier()
    pltpu.sync_copy(hacc0.at[:, pl.ds(base_n, SLICE)], colbuf)
    reduce_cols(colbuf, tmp_v)

    @plsc.parallel_loop(0, SLICE // 16, unroll=2)
    def _(k):
        deg = tmp_v[pl.ds(k * 16, 16)] + 1.0
        i32 = plsc.bitcast(deg, jnp.int32)
        i32 = jnp.int32(0x5F3759DF) - lax.shift_right_logical(i32, 1)
        y = plsc.bitcast(i32, jnp.float32)
        half = deg * 0.5
        for _ in range(3):
            y = y * (1.5 - half * y * y)
        tdinv[pl.ds(k * 16, 16)] = y
        twx[pl.ds(k * 16, 16)] = y * x_sv[pl.ds(k * 16, 16)]

    @pl.when(tid < FULL_T)
    def _():
        pltpu.sync_copy(tdinv, hb2.at[pl.ds(base_n, SLICE)])
        pltpu.sync_copy(twx, hb2.at[pl.ds(N + base_n, SLICE)])

    @pl.when(tid == FULL_T)
    def _():
        pltpu.sync_copy(tdinv.at[pl.ds(0, REM)], hb2.at[pl.ds(base_n, REM)])
        pltpu.sync_copy(twx.at[pl.ds(0, REM)], hb2.at[pl.ds(N + base_n, REM)])

    plsc.subcore_barrier()
    cp_tab = pltpu.async_copy(hb2, tab, sem)
    zero(acc, 0, 2 * NPAD)
    cp_tab.wait()

    n16 = jnp.full((16,), N, jnp.int32)
    npad16 = jnp.full((16,), NPAD, jnp.int32)

    @plsc.parallel_loop(0, EPT // 16, unroll=4)
    def _(i):
        s16 = src_v[pl.ds(i * 16, 16)]
        d16 = dst_v[pl.ds(i * 16, 16)]
        g_d = plsc.load_gather(tab, [s16])
        g_wx = plsc.load_gather(tab, [s16 + n16])
        plsc.addupdate_scatter(acc, [d16], g_wx)
        plsc.addupdate_scatter(acc, [d16 + npad16], g_d)

    pltpu.sync_copy(acc, hacc2.at[tid])
    plsc.subcore_barrier()
    pltpu.sync_copy(hacc2.at[:, pl.ds(base_n, SLICE)], colbuf)
    reduce_cols(colbuf, tmp_v)

    @plsc.parallel_loop(0, SLICE // 16, unroll=2)
    def _(k):
        dv = tdinv[pl.ds(k * 16, 16)]
        s1 = dv * (tmp_v[pl.ds(k * 16, 16)] + twx[pl.ds(k * 16, 16)])
        tz[pl.ds(k * 16, 16)] = dv * s1

    @pl.when(tid < FULL_T)
    def _():
        pltpu.sync_copy(tz, hbz.at[pl.ds(base_n, SLICE)])

    @pl.when(tid == FULL_T)
    def _():
        pltpu.sync_copy(tz.at[pl.ds(0, REM)], hbz.at[pl.ds(base_n, REM)])

    plsc.subcore_barrier()
    cp_tz = pltpu.async_copy(hbz, tab_z, sem)
    cp_t = pltpu.async_copy(hacc2.at[:, pl.ds(NPAD + base_n, SLICE)],
                            colbuf, sem)
    zero(acc, 0, NPAD)
    cp_t.wait()
    reduce_cols(colbuf, tmp_v)

    @plsc.parallel_loop(0, SLICE // 16, unroll=2)
    def _(k):
        dv = tdinv[pl.ds(k * 16, 16)]
        tmp_v[pl.ds(k * 16, 16)] = dv * (tmp_v[pl.ds(k * 16, 16)] + dv)

    pltpu.sync_copy(tmp_v, hst2.at[pl.ds(base_n, SLICE)])

    cp_tz.wait()

    @plsc.parallel_loop(0, EPT // 16, unroll=4)
    def _(i):
        s16 = src_v[pl.ds(i * 16, 16)]
        d16 = dst_v[pl.ds(i * 16, 16)]
        g_z = plsc.load_gather(tab_z, [s16])
        plsc.addupdate_scatter(acc, [d16], g_z)

    pltpu.sync_copy(acc.at[pl.ds(0, NPAD)], hacc0.at[tid])
    plsc.subcore_barrier()
    pltpu.sync_copy(hacc0.at[:, pl.ds(base_n, SLICE)], colbuf)
    reduce_cols(colbuf, tmp_v)

    @plsc.parallel_loop(0, SLICE // 16, unroll=2)
    def _(k):
        dv = tdinv[pl.ds(k * 16, 16)]
        tmp_v[pl.ds(k * 16, 16)] = dv * (tmp_v[pl.ds(k * 16, 16)]
                                         + tz[pl.ds(k * 16, 16)])

    pltpu.sync_copy(tmp_v, hst2.at[pl.ds(NPAD + base_n, SLICE)])
    plsc.subcore_barrier()

    @pl.when(tid < ONT)
    def _():
        ob = tid * OCHUNK
        pltpu.sync_copy(hst2.at[pl.ds(ob, OCHUNK)], stage_v)
        to2d(stage_v, tmp2)
        pltpu.sync_copy(tmp2, t_h.at[pl.ds(tid * (OCHUNK // 128), 8), :])
        pltpu.sync_copy(hst2.at[pl.ds(NPAD + ob, OCHUNK)], stage_v)
        to2d(stage_v, tmp2)
        pltpu.sync_copy(tmp2, s2_h.at[pl.ds(tid * (OCHUNK // 128), 8), :])


_sc_graph = pl.kernel(
    _sc_body,
    out_type=(
        jax.ShapeDtypeStruct((NPAD // 128, 128), jnp.float32),
        jax.ShapeDtypeStruct((NPAD // 128, 128), jnp.float32),
        jax.ShapeDtypeStruct((NS, NPAD), jnp.float32),
        jax.ShapeDtypeStruct((NS, 2 * NPAD), jnp.float32),
        jax.ShapeDtypeStruct((2 * N,), jnp.float32),
        jax.ShapeDtypeStruct((N,), jnp.float32),
        jax.ShapeDtypeStruct((2 * NPAD,), jnp.float32),
    ),
    mesh=plsc.VectorSubcoreMesh(
        core_axis_name="c", subcore_axis_name="s", num_cores=1,
        num_subcores=NS),
    compiler_params=pltpu.CompilerParams(needs_layout_passes=False),
    scratch_types=[
        pltpu.VMEM((EPT,), jnp.int32),
        pltpu.VMEM((EPT,), jnp.int32),
        pltpu.VMEM((2 * N,), jnp.float32),
        pltpu.VMEM((N,), jnp.float32),
        pltpu.VMEM((SLICE,), jnp.float32),
        pltpu.VMEM((2 * NPAD,), jnp.float32),
        pltpu.VMEM((NS, SLICE), jnp.float32),
        pltpu.VMEM((SLICE,), jnp.float32),
        pltpu.VMEM((SLICE,), jnp.float32),
        pltpu.VMEM((SLICE,), jnp.float32),
        pltpu.VMEM((SLICE,), jnp.float32),
        pltpu.VMEM((OCHUNK // 128, 128), jnp.float32),
        pltpu.VMEM((OCHUNK,), jnp.float32),
        pltpu.SemaphoreType.DMA,
    ],
)


def _uc_body(w1_ref, b1_ref, W2_ref, u_ref, c_ref):
    u_ref[...] = jnp.dot(w1_ref[...], W2_ref[...],
                         preferred_element_type=jnp.float32)
    c_ref[...] = jnp.dot(b1_ref[...], W2_ref[...],
                         preferred_element_type=jnp.float32)


_uc = pl.pallas_call(
    _uc_body,
    out_shape=(
        jax.ShapeDtypeStruct((1, H2), jnp.float32),
        jax.ShapeDtypeStruct((1, H2), jnp.float32),
    ),
)


ROWS_BLK = 2048
RB8 = ROWS_BLK // 128


def _main_body(s2f_ref, tf_ref, u_ref, c_ref, gamma_ref, beta_ref, l1w_ref,
               l1b_ref, l2w_ref, l2b_ref, s2_ref, t_ref, o_ref,
               p_s, q_s, r_s):
    i = pl.program_id(0)

    @pl.when(i == 0)
    def _():
        rows = lax.broadcasted_iota(jnp.int32, (NPAD // 128, 128), 0)
        cols = lax.broadcasted_iota(jnp.int32, (NPAD // 128, 128), 1)
        mask = (rows * 128 + cols) < N

        s2 = jnp.where(mask, s2f_ref[...], 0.0)
        t = jnp.where(mask, tf_ref[...], 0.0)
        inv_n = 1.0 / N
        m_s = jnp.sum(s2) * inv_n
        m_t = jnp.sum(t) * inv_n
        ds = jnp.where(mask, s2 - m_s, 0.0)
        dt = jnp.where(mask, t - m_t, 0.0)
        vs = jnp.sum(ds * ds) * inv_n
        vt = jnp.sum(dt * dt) * inv_n
        cv = jnp.sum(ds * dt) * inv_n

        u = u_ref[...]
        c = c_ref[...]
        var = vs * u * u + vt * c * c + 2.0 * cv * u * c
        scale = gamma_ref[...] / jnp.sqrt(var + EPS)

        p = jnp.dot(u * scale, l1w_ref[...],
                    preferred_element_type=jnp.float32)
        q = jnp.dot(c * scale, l1w_ref[...],
                    preferred_element_type=jnp.float32)
        r = jnp.dot(beta_ref[...], l1w_ref[...],
                    preferred_element_type=jnp.float32) + l1b_ref[...]
        p_s[...] = p
        q_s[...] = q
        r_s[...] = r - m_s * p - m_t * q

    na = lax.broadcasted_iota(jnp.int32, (ROWS_BLK, RB8), 0)
    ka = lax.broadcasted_iota(jnp.int32, (ROWS_BLK, RB8), 1)
    A = jnp.where(lax.shift_right_logical(na, 7) == ka, 1.0, 0.0)
    nd = lax.broadcasted_iota(jnp.int32, (ROWS_BLK, 128), 0)
    cd = lax.broadcasted_iota(jnp.int32, (ROWS_BLK, 128), 1)
    Dm = jnp.where((nd & 127) == cd, 1.0, 0.0)
    gs = jnp.dot(A, s2_ref[...], preferred_element_type=jnp.float32)
    gt = jnp.dot(A, t_ref[...], preferred_element_type=jnp.float32)
    s2c = jnp.sum(gs * Dm, axis=1, keepdims=True)
    tc = jnp.sum(gt * Dm, axis=1, keepdims=True)
    h = s2c * p_s[...] + tc * q_s[...] + r_s[...]
    h = jnp.where(h > 0, h, NEG_SLOPE * h)
    logits = jnp.dot(h, l2w_ref[...],
                     preferred_element_type=jnp.float32) + l2b_ref[...]
    m = jnp.max(logits, axis=1, keepdims=True)
    z = logits - m
    lse = jnp.log(jnp.sum(jnp.exp(z), axis=1, keepdims=True))
    o_ref[...] = z - lse


_main = pl.pallas_call(
    _main_body,
    grid=((N + ROWS_BLK - 1) // ROWS_BLK,),
    in_specs=[
        pl.BlockSpec((NPAD // 128, 128), lambda i: (0, 0)),
        pl.BlockSpec((NPAD // 128, 128), lambda i: (0, 0)),
        pl.BlockSpec((1, H2), lambda i: (0, 0)),
        pl.BlockSpec((1, H2), lambda i: (0, 0)),
        pl.BlockSpec((1, H2), lambda i: (0, 0)),
        pl.BlockSpec((1, H2), lambda i: (0, 0)),
        pl.BlockSpec((H2, H3), lambda i: (0, 0)),
        pl.BlockSpec((1, H3), lambda i: (0, 0)),
        pl.BlockSpec((H3, OUT), lambda i: (0, 0)),
        pl.BlockSpec((1, OUT), lambda i: (0, 0)),
        pl.BlockSpec((RB8, 128), lambda i: (i, 0)),
        pl.BlockSpec((RB8, 128), lambda i: (i, 0)),
    ],
    out_specs=pl.BlockSpec((ROWS_BLK, OUT), lambda i: (i, 0)),
    out_shape=jax.ShapeDtypeStruct((N, OUT), jnp.float32),
    scratch_shapes=[
        pltpu.VMEM((1, H3), jnp.float32),
        pltpu.VMEM((1, H3), jnp.float32),
        pltpu.VMEM((1, H3), jnp.float32),
    ],
)


def kernel(x, edge_index, W1, b1, W2, b2, gamma, beta, lin1_W, lin1_b,
           lin2_W, lin2_b):
    del b2
    xf = x.reshape(N).astype(jnp.float32)

    u, c = _uc(W1.reshape(1, H1), b1.reshape(1, H1), W2)
    s2p, tp, _, _, _, _, _ = _sc_graph(edge_index.reshape(2 * E), xf)

    return _main(
        s2p,
        tp,
        u,
        c,
        gamma.reshape(1, H2),
        beta.reshape(1, H2),
        lin1_W,
        lin1_b.reshape(1, H3),
        lin2_W,
        lin2_b.reshape(1, OUT),
        s2p,
        tp,
    )
